# baseline (speedup 1.0000x reference)
import functools

import jax
import jax.numpy as jnp
from jax import lax
from jax.experimental import pallas as pl
from jax.experimental.pallas import tpu as pltpu
from jax.experimental.pallas import tpu_sc as plsc

NC = 2
NS = 16
L = 16

N_NODES = 10000
N_PAD_NODES = 10240
EMB = 128
CHUNK = 128
G = 2048

_NO_LAYOUT = pltpu.CompilerParams(needs_layout_passes=False)


def _rsqrt_newton(d):
    xi = lax.bitcast_convert_type(d, jnp.int32)
    xi = 0x5F3759DF - lax.shift_right_logical(xi, 1)
    y = lax.bitcast_convert_type(xi, jnp.float32)
    for _ in range(3):
        y = y * (1.5 - 0.5 * d * y * y)
    return y


def _sc_factors(uidx, iidx, w, e_pad):
    ept16 = e_pad // NS
    mesh = plsc.VectorSubcoreMesh(core_axis_name="c", subcore_axis_name="s")

    @functools.partial(
        pl.kernel,
        out_type=jax.ShapeDtypeStruct((NC, e_pad // CHUNK, CHUNK), jnp.float32),
        mesh=mesh,
        compiler_params=_NO_LAYOUT,
        scratch_types=[
            pltpu.VMEM((ept16 // CHUNK, CHUNK), jnp.int32),
            pltpu.VMEM((ept16 // CHUNK, CHUNK), jnp.float32),
            pltpu.VMEM((ept16 // CHUNK, CHUNK), jnp.float32),
            pltpu.VMEM((80, EMB), jnp.float32),
            pltpu.VMEM((80,), jnp.int32),
            pltpu.VMEM((8, EMB), jnp.float32),
            pltpu.VMEM_SHARED((80, EMB), jnp.float32),
        ],
    )
    def ka(idx_h, w_h, out_h, gidx, wloc, fout, hist, rowids, tbl, deg):
        c = lax.axis_index("c")
        s = lax.axis_index("s")
        pos16 = lax.iota(jnp.int32, L)
        zeros16 = jnp.zeros((L,), jnp.float32)
        nck = ept16 // CHUNK
        c0 = pl.multiple_of((s * ept16) // CHUNK, 8)

        pltpu.sync_copy(idx_h.at[c, pl.ds(c0, nck)], gidx)
        pltpu.sync_copy(w_h.at[c, pl.ds(c0, nck)], wloc)

        def zero_body(r, _):
            for j in range(EMB // L):
                hist[r, pl.ds(j * L, L)] = zeros16
            return 0

        lax.fori_loop(0, 80, zero_body, 0)
        for r in range(5):
            rowids[pl.ds(r * L, L)] = pos16 + r * L
        for j in range(EMB // L):
            for r in range(5):
                tbl[r, pl.ds(j * L, L)] = zeros16
        pltpu.sync_copy(tbl.at[pl.ds(0, 5)], deg.at[pl.ds(s * 5, 5)])
        plsc.subcore_barrier()

        def chunk_body(kk, _):
            for g in range(CHUNK // L):
                v = gidx[kk, pl.ds(g * L, L)]
                cnt, last = plsc.scan_count(v)
                plsc.addupdate_scatter(
                    hist,
                    [lax.shift_right_logical(v, 7), v & (EMB - 1)],
                    cnt.astype(jnp.float32), mask=last)
            return 0

        lax.fori_loop(0, nck, chunk_body, 0)
        pltpu.sync_copy(hist, deg.at[rowids], add=True)
        plsc.subcore_barrier()

        pltpu.sync_copy(deg.at[pl.ds(s * 5, 5)], tbl.at[pl.ds(0, 5)])

        def rsq_body(r, _):
            for j in range(EMB // L):
                d = jnp.maximum(tbl[r, pl.ds(j * L, L)], 1.0)
                tbl[r, pl.ds(j * L, L)] = _rsqrt_newton(d)
            return 0

        lax.fori_loop(0, 5, rsq_body, 0)
        pltpu.sync_copy(tbl.at[pl.ds(0, 5)], deg.at[pl.ds(s * 5, 5)])
        plsc.subcore_barrier()

        pltpu.sync_copy(deg, hist)

        def fac_body(kk, _):
            for g in range(CHUNK // L):
                v = gidx[kk, pl.ds(g * L, L)]
                f = plsc.load_gather(
                    hist,
                    [lax.shift_right_logical(v, 7), v & (EMB - 1)])
                wv = wloc[kk, pl.ds(g * L, L)]
                fout[kk, pl.ds(g * L, L)] = f * wv
            return 0

        lax.fori_loop(0, nck, fac_body, 0)
        pltpu.sync_copy(fout, out_h.at[c, pl.ds(c0, nck)])

    idx_stack = jnp.stack([uidx, iidx]).reshape(NC, e_pad // CHUNK, CHUNK)
    ones = jnp.ones_like(w)
    w_stack = jnp.stack([w, ones]).reshape(NC, e_pad // CHUNK, CHUNK)
    return ka(idx_stack, w_stack)


def _sc_messages(gidx_all, didx_all, ew2d, emb_stack, e_pad):
    ept = e_pad // NS
    ngrp = ept // G
    npair = G // CHUNK // 2
    nck = G // CHUNK
    mesh = plsc.VectorSubcoreMesh(core_axis_name="c", subcore_axis_name="s")

    @functools.partial(
        pl.kernel,
        out_type=jax.ShapeDtypeStruct((NC, N_PAD_NODES, EMB), jnp.float32),
        mesh=mesh,
        compiler_params=pltpu.CompilerParams(
            needs_layout_passes=False, use_tc_tiling_on_sc=False),
        scratch_types=[
            pltpu.VMEM((G // CHUNK, CHUNK), jnp.int32),
            pltpu.VMEM((G // CHUNK, CHUNK), jnp.int32),
            pltpu.VMEM((G // CHUNK, CHUNK), jnp.float32),
            pltpu.VMEM((CHUNK, EMB // 2), jnp.int32),
            pltpu.VMEM((CHUNK, EMB // 2), jnp.int32),
            pltpu.VMEM((CHUNK, EMB), jnp.float32),
            pltpu.VMEM_SHARED((N_PAD_NODES, EMB), jnp.float32),
            pltpu.SemaphoreType.DMA,
            pltpu.SemaphoreType.DMA,
        ],
    )
    def kb(gidx_h, didx_h, ew_h, emb_h, out_h,
           gl, dl, ewl, rows0, rows1, rowsf, acc, sem0, sem1):
        c = lax.axis_index("c")
        s = lax.axis_index("s")
        zeros16 = jnp.zeros((L,), jnp.float32)
        bufs = ((rows0, sem0), (rows1, sem1))

        def start_gather(kk, b):
            rows, sem = bufs[b]
            pltpu.async_copy(emb_h.at[gl.at[kk]], rows, sem)

        def drain_gather(b):
            rows, sem = bufs[b]
            pltpu.make_async_copy(emb_h.at[gl.at[0]], rows, sem).wait()


        def scale(kk, b):
            rows, sem = bufs[b]

            def scale_body(g2, _):
                evec = ewl[kk, pl.ds(g2 * L, L)]
                for l in range(L):
                    sc = evec[l]
                    e = g2 * L + l
                    for q in range(EMB // (2 * L)):
                        m32 = rows[e, pl.ds(q * L, L)]
                        mb = plsc.bitcast(m32, jnp.bfloat16)
                        a, b2 = plsc.unpack(
                            mb, format=plsc.PackFormat.INTERLEAVED)
                        rowsf[e, pl.ds(q * 2 * L, L)] = a * sc
                        rowsf[e, pl.ds(q * 2 * L + L, L)] = b2 * sc
                return 0

            lax.fori_loop(0, CHUNK // L, scale_body, 0)

        def zero_body(r, _):
            for j in range(EMB // L):
                rowsf[r, pl.ds(j * L, L)] = zeros16
            return 0

        lax.fori_loop(0, CHUNK, zero_body, 0)
        for m in range(5):
            pltpu.sync_copy(rowsf, acc.at[pl.ds((s * 5 + m) * CHUNK, CHUNK)])
        plsc.subcore_barrier()

        def msg_group(g_id, _):
            off = s * ept + g_id * G
            c0 = pl.multiple_of(off // CHUNK, 8)
            pltpu.sync_copy(gidx_h.at[c, pl.ds(c0, nck)], gl)
            pltpu.sync_copy(didx_h.at[c, pl.ds(c0, nck)], dl)
            pltpu.sync_copy(ew_h.at[pl.ds(c0, nck)], ewl)
            start_gather(0, 0)

            def pair_body(k2, _):
                drain_gather(0)
                start_gather(2 * k2 + 1, 1)
                scale(2 * k2, 0)
                pltpu.sync_copy(rowsf, acc.at[dl.at[2 * k2]], add=True)
                drain_gather(1)

                @pl.when(k2 < npair - 1)
                def _():
                    start_gather(2 * k2 + 2, 0)

                scale(2 * k2 + 1, 1)
                pltpu.sync_copy(rowsf, acc.at[dl.at[2 * k2 + 1]], add=True)
                return 0

            lax.fori_loop(0, npair, pair_body, 0)
            return 0

        lax.fori_loop(0, ngrp, msg_group, 0)
        plsc.subcore_barrier()

        for m in range(5):
            r0 = (s * 5 + m) * CHUNK
            pltpu.sync_copy(
                acc.at[pl.ds(r0, CHUNK)],
                out_h.at[c, pl.ds(pl.multiple_of(r0, 8), CHUNK)])

    return kb(gidx_all, didx_all, ew2d, emb_stack)


def _mm_relu_body(msg_ref, emb_ref, w_ref, out_ref):
    x = msg_ref[0] + emb_ref[0]
    y = lax.dot_general(
        x, w_ref[0],
        dimension_numbers=(((1,), (1,)), ((), ())),
        preferred_element_type=jnp.float32,
    )
    out_ref[0] = jnp.maximum(y, 0.0)


def _mm_relu(msg, emb, w):
    n = msg.shape[1]
    br = 2000
    return pl.pallas_call(
        _mm_relu_body,
        grid=(2, n // br),
        in_specs=[
            pl.BlockSpec((1, br, 128), lambda g, r: (g, r, 0)),
            pl.BlockSpec((1, br, 128), lambda g, r: (g, r, 0)),
            pl.BlockSpec((1, 128, 128), lambda g, r: (g, 0, 0)),
        ],
        out_specs=pl.BlockSpec((1, br, 128), lambda g, r: (g, r, 0)),
        out_shape=jax.ShapeDtypeStruct((2, n, 128), jnp.float32),
    )(msg, emb, w)


def kernel(u_emb, i_emb, edge_index, weights, W_u, W_i):
    e = edge_index.shape[1]
    per_round = G * NC * NS
    e_pad = -(-e // per_round) * per_round
    n_pad = e_pad - e

    uidx = edge_index[0]
    iidx = edge_index[1]
    if n_pad:
        pad_nodes = N_NODES + (
            jnp.arange(n_pad, dtype=jnp.int32) % (N_PAD_NODES - N_NODES))
        uidx = jnp.concatenate([uidx, pad_nodes])
        iidx = jnp.concatenate([iidx, pad_nodes])
        weights = jnp.concatenate([weights, jnp.zeros((n_pad,), jnp.float32)])

    pwb = _sc_factors(uidx, iidx, weights, e_pad)
    ew2d = pwb[0] * pwb[1]

    shape3 = (e_pad // CHUNK, CHUNK)
    gidx_all = jnp.stack(
        [iidx.reshape(shape3) + N_PAD_NODES, uidx.reshape(shape3)])
    didx_all = jnp.stack([uidx.reshape(shape3), iidx.reshape(shape3)])

    zpad = jnp.zeros((N_PAD_NODES - N_NODES, EMB), jnp.float32)
    emb_stack = jnp.concatenate([u_emb, zpad, i_emb, zpad])
    n_all = emb_stack.shape[0]
    emb_stack = (
        emb_stack.reshape(n_all, EMB // 32, 2, 16)
        .transpose(0, 1, 3, 2)
        .reshape(n_all, EMB // 2, 2)
        .astype(jnp.bfloat16))
    emb_stack = lax.bitcast_convert_type(emb_stack, jnp.int32)

    msg = _sc_messages(gidx_all, didx_all, ew2d, emb_stack, e_pad)
    msg = msg[:, :N_NODES]

    emb = jnp.stack([u_emb, i_emb])
    w = jnp.stack([W_u, W_i])
    out = _mm_relu(msg, emb, w)
    return (out[0], out[1])
